# baseline (speedup 1.0000x reference)
import functools

import jax
import jax.numpy as jnp
from jax import lax
from jax.experimental import pallas as pl
from jax.experimental.pallas import tpu as pltpu
from jax.experimental.pallas import tpu_sc as plsc

D_MODEL = 128
HALF = 64
_NW = 32
_CHUNK = 128


def _transform_body(x_ref, p_ref, w_ref, b_ref, oe_ref, op_ref):
    w1 = w_ref[:, 0:HALF]
    oe_ref[:] = lax.dot_general(x_ref[:], w1, (((1,), (1,)), ((), ())),
                                preferred_element_type=jnp.float32)

    @pl.when(pl.program_id(0) == 0)
    def _prop():
        w2 = w_ref[:, HALF:D_MODEL]
        op_ref[:] = lax.dot_general(p_ref[:], w2, (((1,), (1,)), ((), ())),
                                    preferred_element_type=jnp.float32
                                    ) + b_ref[:]


def _transform_tables(etab, ptab, w, b2d):
    v = etab.shape[0]
    vp = ptab.shape[0]
    r = 2000
    return pl.pallas_call(
        _transform_body,
        grid=(v // r,),
        in_specs=[pl.BlockSpec((r, HALF), lambda i: (i, 0)),
                  pl.BlockSpec((vp, HALF), lambda i: (0, 0)),
                  pl.BlockSpec((D_MODEL, D_MODEL), lambda i: (0, 0)),
                  pl.BlockSpec((1, D_MODEL), lambda i: (0, 0))],
        out_specs=[pl.BlockSpec((r, D_MODEL), lambda i: (i, 0)),
                   pl.BlockSpec((vp, D_MODEL), lambda i: (0, 0))],
        out_shape=[jax.ShapeDtypeStruct((v, D_MODEL), jnp.float32),
                   jax.ShapeDtypeStruct((vp, D_MODEL), jnp.float32)],
    )(etab, ptab, w, b2d)


def _sc_combine(te, tp, eid, pid):
    b = eid.shape[0]
    per_w = b // _NW
    n_iter = per_w // _CHUNK
    mesh = plsc.VectorSubcoreMesh(core_axis_name="c", subcore_axis_name="s")

    assert n_iter % 4 == 0 and n_iter >= 8

    @functools.partial(
        pl.kernel, mesh=mesh,
        out_type=jax.ShapeDtypeStruct((b, D_MODEL), jnp.float32),
        scratch_types=[
            pltpu.VMEM((4, _CHUNK), jnp.int32),
            pltpu.VMEM((4, _CHUNK), jnp.int32),
            pltpu.VMEM((4, _CHUNK, D_MODEL), jnp.float32),
            pltpu.VMEM_SHARED((1000, D_MODEL), jnp.float32),
            pltpu.SemaphoreType.DMA((4,)),
            pltpu.SemaphoreType.DMA((4,)),
            pltpu.SemaphoreType.DMA((4,)),
            pltpu.SemaphoreType.DMA((4,)),
        ],
    )
    def k(te_hbm, tp_hbm, eid_hbm, pid_hbm, out_hbm, eixb, pixb, ebufs,
          tp_sh, isem, esem, psem, wsem):
        wid = lax.axis_index("s") * 2 + lax.axis_index("c")
        base = wid * per_w

        @pl.when(lax.axis_index("s") == 0)
        def _stage():
            pltpu.sync_copy(tp_hbm, tp_sh)

        plsc.subcore_barrier()

        for i in (0, 1):
            off = base + i * _CHUNK
            pltpu.sync_copy(eid_hbm.at[pl.ds(off, _CHUNK)], eixb.at[i])
            pltpu.sync_copy(pid_hbm.at[pl.ds(off, _CHUNK)], pixb.at[i])
        for i in (2, 3):
            off = base + i * _CHUNK
            pltpu.async_copy(eid_hbm.at[pl.ds(off, _CHUNK)], eixb.at[i],
                             isem.at[i])
            pltpu.async_copy(pid_hbm.at[pl.ds(off, _CHUNK)], pixb.at[i],
                             isem.at[i])
        for i in (0, 1):
            pltpu.async_copy(te_hbm.at[eixb.at[i]], ebufs.at[i], esem.at[i])
        pltpu.make_async_copy(te_hbm.at[eixb.at[0]], ebufs.at[0],
                              esem.at[0]).wait()
        pltpu.async_copy(tp_sh.at[pixb.at[0]], ebufs.at[0], psem.at[0],
                         add=True)

        def quad(fq, carry):
            for sub in range(4):
                i = fq * 4 + sub
                s0 = sub
                s1 = (sub + 1) % 4
                s2 = (sub + 2) % 4

                @pl.when(i + 1 < n_iter)
                def _padd():
                    pltpu.make_async_copy(te_hbm.at[eixb.at[s1]],
                                          ebufs.at[s1], esem.at[s1]).wait()
                    pltpu.async_copy(tp_sh.at[pixb.at[s1]], ebufs.at[s1],
                                     psem.at[s1], add=True)

                pltpu.make_async_copy(tp_sh.at[pixb.at[s0]], ebufs.at[s0],
                                      psem.at[s0]).wait()
                off = base + i * _CHUNK
                pltpu.async_copy(ebufs.at[s0], out_hbm.at[pl.ds(off, _CHUNK)],
                                 wsem.at[s0])

                @pl.when(i >= 2)
                def _wdrain():
                    pltpu.make_async_copy(
                        ebufs.at[s2], out_hbm.at[pl.ds(base, _CHUNK)],
                        wsem.at[s2]).wait()

                @pl.when(i + 2 < n_iter)
                def _gnext():
                    pltpu.make_async_copy(
                        eid_hbm.at[pl.ds(base, _CHUNK)], eixb.at[s2],
                        isem.at[s2]).wait()
                    pltpu.make_async_copy(
                        pid_hbm.at[pl.ds(base, _CHUNK)], pixb.at[s2],
                        isem.at[s2]).wait()
                    pltpu.async_copy(te_hbm.at[eixb.at[s2]], ebufs.at[s2],
                                     esem.at[s2])

                @pl.when(i + 4 < n_iter)
                def _pfidx():
                    off4 = base + (i + 4) * _CHUNK
                    pltpu.async_copy(eid_hbm.at[pl.ds(off4, _CHUNK)],
                                     eixb.at[s0], isem.at[s0])
                    pltpu.async_copy(pid_hbm.at[pl.ds(off4, _CHUNK)],
                                     pixb.at[s0], isem.at[s0])
            return carry

        lax.fori_loop(0, n_iter // 4, quad, 0)

        for i in (n_iter - 2, n_iter - 1):
            pltpu.make_async_copy(ebufs.at[i % 4],
                                  out_hbm.at[pl.ds(base, _CHUNK)],
                                  wsem.at[i % 4]).wait()

    return k(te, tp, eid, pid)


def kernel(element_ids, property_ids, element_table, property_table,
           fusion_w, fusion_b):
    bsz, n = element_ids.shape
    eid = element_ids.reshape(-1).astype(jnp.int32)
    pid = property_ids.reshape(-1).astype(jnp.int32)
    te, tp = _transform_tables(element_table, property_table, fusion_w,
                               fusion_b.reshape(1, -1))
    out = _sc_combine(te, tp, eid, pid)
    return out.reshape(bsz, n, D_MODEL)


# speedup vs baseline: 1.0752x; 1.0752x over previous
import functools

import jax
import jax.numpy as jnp
from jax import lax
from jax.experimental import pallas as pl
from jax.experimental.pallas import tpu as pltpu
from jax.experimental.pallas import tpu_sc as plsc

D_MODEL = 128
HALF = 64
_NW = 32
_CHUNK = 128


def _transform_body(x_ref, p_ref, w_ref, b_ref, oe_ref, op_ref):
    w1 = w_ref[:, 0:HALF]
    oe_ref[:] = lax.dot_general(x_ref[:], w1, (((1,), (1,)), ((), ())),
                                preferred_element_type=jnp.float32)

    @pl.when(pl.program_id(0) == 0)
    def _prop():
        w2 = w_ref[:, HALF:D_MODEL]
        op_ref[:] = lax.dot_general(p_ref[:], w2, (((1,), (1,)), ((), ())),
                                    preferred_element_type=jnp.float32
                                    ) + b_ref[:]


def _transform_tables(etab, ptab, w, b2d):
    v = etab.shape[0]
    vp = ptab.shape[0]
    r = 2000
    return pl.pallas_call(
        _transform_body,
        grid=(v // r,),
        in_specs=[pl.BlockSpec((r, HALF), lambda i: (i, 0)),
                  pl.BlockSpec((vp, HALF), lambda i: (0, 0)),
                  pl.BlockSpec((D_MODEL, D_MODEL), lambda i: (0, 0)),
                  pl.BlockSpec((1, D_MODEL), lambda i: (0, 0))],
        out_specs=[pl.BlockSpec((r, D_MODEL), lambda i: (i, 0)),
                   pl.BlockSpec((vp, D_MODEL), lambda i: (0, 0))],
        out_shape=[jax.ShapeDtypeStruct((v, D_MODEL), jnp.float32),
                   jax.ShapeDtypeStruct((vp, D_MODEL), jnp.float32)],
    )(etab, ptab, w, b2d)


def _sc_combine(te, tp, eid, pid):
    b = eid.shape[0]
    per_w = b // _NW
    n_iter = per_w // _CHUNK
    mesh = plsc.VectorSubcoreMesh(core_axis_name="c", subcore_axis_name="s")

    assert n_iter % 4 == 0 and n_iter >= 8

    @functools.partial(
        pl.kernel, mesh=mesh,
        out_type=jax.ShapeDtypeStruct((b, D_MODEL), jnp.float32),
        scratch_types=[
            pltpu.VMEM((2, 4 * _CHUNK), jnp.int32),
            pltpu.VMEM((2, 4 * _CHUNK), jnp.int32),
            pltpu.VMEM((2, _CHUNK, D_MODEL), jnp.float32),
            pltpu.VMEM((2, _CHUNK, D_MODEL), jnp.float32),
            pltpu.VMEM((2, _CHUNK, D_MODEL), jnp.float32),
            pltpu.VMEM_SHARED((1000, D_MODEL), jnp.float32),
            pltpu.SemaphoreType.DMA((2,)),
            pltpu.SemaphoreType.DMA((2,)),
            pltpu.SemaphoreType.DMA((2,)),
            pltpu.SemaphoreType.DMA((2,)),
        ],
    )
    def k(te_hbm, tp_hbm, eid_hbm, pid_hbm, out_hbm, eixb, pixb, ebufs,
          pbufs, obufs, tp_sh, isem, esem, psem, wsem):
        wid = lax.axis_index("s") * 2 + lax.axis_index("c")
        base = wid * per_w

        @pl.when(lax.axis_index("s") == 0)
        def _stage():
            pltpu.sync_copy(tp_hbm, tp_sh)

        plsc.subcore_barrier()

        pltpu.sync_copy(eid_hbm.at[pl.ds(base, 4 * _CHUNK)], eixb.at[0])
        pltpu.sync_copy(pid_hbm.at[pl.ds(base, 4 * _CHUNK)], pixb.at[0])
        for i in (0, 1):
            sl = pl.ds(i * _CHUNK, _CHUNK)
            pltpu.async_copy(te_hbm.at[eixb.at[0, sl]], ebufs.at[i],
                             esem.at[i])
            pltpu.async_copy(tp_sh.at[pixb.at[0, sl]], pbufs.at[i],
                             psem.at[i])
        pltpu.async_copy(eid_hbm.at[pl.ds(base + 4 * _CHUNK, 4 * _CHUNK)],
                         eixb.at[1], isem.at[1])
        pltpu.async_copy(pid_hbm.at[pl.ds(base + 4 * _CHUNK, 4 * _CHUNK)],
                         pixb.at[1], isem.at[1])

        def quad(fq, carry):
            cur = fq % 2
            nxt = 1 - cur
            for sub in range(4):
                i = fq * 4 + sub
                bb = sub % 2
                sq = pl.ds(sub * _CHUNK, _CHUNK)

                pltpu.make_async_copy(te_hbm.at[eixb.at[cur, sq]],
                                      ebufs.at[bb], esem.at[bb]).wait()
                pltpu.make_async_copy(tp_sh.at[pixb.at[cur, sq]],
                                      pbufs.at[bb], psem.at[bb]).wait()

                @pl.when(i >= 2)
                def _wdrain():
                    pltpu.make_async_copy(
                        obufs.at[bb], out_hbm.at[pl.ds(base, _CHUNK)],
                        wsem.at[bb]).wait()

                if sub == 3:
                    @pl.when(i + 5 < n_iter)
                    def _pfidx():
                        offb = base + (fq + 2) * 4 * _CHUNK
                        pltpu.async_copy(
                            eid_hbm.at[pl.ds(offb, 4 * _CHUNK)],
                            eixb.at[cur], isem.at[cur])
                        pltpu.async_copy(
                            pid_hbm.at[pl.ds(offb, 4 * _CHUNK)],
                            pixb.at[cur], isem.at[cur])

                def addrow(r4, c2):
                    for rr in range(4):
                        r2 = r4 * 4 + rr
                        for j in range(D_MODEL // 16):
                            sl = pl.ds(j * 16, 16)
                            obufs[bb, r2, sl] = (ebufs[bb, r2, sl]
                                                 + pbufs[bb, r2, sl])
                    return c2

                lax.fori_loop(0, _CHUNK // 4, addrow, 0)

                off = base + i * _CHUNK
                pltpu.async_copy(obufs.at[bb], out_hbm.at[pl.ds(off, _CHUNK)],
                                 wsem.at[bb])

                @pl.when(i + 2 < n_iter)
                def _gnext():
                    j2 = i + 2
                    slot2 = nxt if sub >= 2 else cur
                    sq2 = pl.ds(((sub + 2) % 4) * _CHUNK, _CHUNK)
                    if sub == 2:
                        pltpu.make_async_copy(
                            eid_hbm.at[pl.ds(base, 4 * _CHUNK)],
                            eixb.at[slot2], isem.at[slot2]).wait()
                        pltpu.make_async_copy(
                            pid_hbm.at[pl.ds(base, 4 * _CHUNK)],
                            pixb.at[slot2], isem.at[slot2]).wait()
                    pltpu.async_copy(te_hbm.at[eixb.at[slot2, sq2]],
                                     ebufs.at[bb], esem.at[bb])
                    pltpu.async_copy(tp_sh.at[pixb.at[slot2, sq2]],
                                     pbufs.at[bb], psem.at[bb])
            return carry

        lax.fori_loop(0, n_iter // 4, quad, 0)

        for bb in (0, 1):
            pltpu.make_async_copy(obufs.at[bb],
                                  out_hbm.at[pl.ds(base, _CHUNK)],
                                  wsem.at[bb]).wait()

    return k(te, tp, eid, pid)


def kernel(element_ids, property_ids, element_table, property_table,
           fusion_w, fusion_b):
    bsz, n = element_ids.shape
    eid = element_ids.reshape(-1).astype(jnp.int32)
    pid = property_ids.reshape(-1).astype(jnp.int32)
    te, tp = _transform_tables(element_table, property_table, fusion_w,
                               fusion_b.reshape(1, -1))
    out = _sc_combine(te, tp, eid, pid)
    return out.reshape(bsz, n, D_MODEL)


# speedup vs baseline: 1.0767x; 1.0015x over previous
import functools

import jax
import jax.numpy as jnp
from jax import lax
from jax.experimental import pallas as pl
from jax.experimental.pallas import tpu as pltpu
from jax.experimental.pallas import tpu_sc as plsc

D_MODEL = 128
HALF = 64
_NW = 32
_CHUNK = 128


def _transform_body(x_ref, p_ref, w_ref, b_ref, oe_ref, op_ref):
    w1 = w_ref[:, 0:HALF]
    oe_ref[:] = lax.dot_general(x_ref[:], w1, (((1,), (1,)), ((), ())),
                                preferred_element_type=jnp.float32)

    @pl.when(pl.program_id(0) == 0)
    def _prop():
        w2 = w_ref[:, HALF:D_MODEL]
        op_ref[:] = lax.dot_general(p_ref[:], w2, (((1,), (1,)), ((), ())),
                                    preferred_element_type=jnp.float32
                                    ) + b_ref[:]


def _transform_tables(etab, ptab, w, b2d):
    v = etab.shape[0]
    vp = ptab.shape[0]
    r = 2000
    return pl.pallas_call(
        _transform_body,
        grid=(v // r,),
        in_specs=[pl.BlockSpec((r, HALF), lambda i: (i, 0)),
                  pl.BlockSpec((vp, HALF), lambda i: (0, 0)),
                  pl.BlockSpec((D_MODEL, D_MODEL), lambda i: (0, 0)),
                  pl.BlockSpec((1, D_MODEL), lambda i: (0, 0))],
        out_specs=[pl.BlockSpec((r, D_MODEL), lambda i: (i, 0)),
                   pl.BlockSpec((vp, D_MODEL), lambda i: (0, 0))],
        out_shape=[jax.ShapeDtypeStruct((v, D_MODEL), jnp.float32),
                   jax.ShapeDtypeStruct((vp, D_MODEL), jnp.float32)],
    )(etab, ptab, w, b2d)


def _sc_combine(te, tp, eid, pid):
    b = eid.shape[0]
    per_w = b // _NW
    n_iter = per_w // _CHUNK
    mesh = plsc.VectorSubcoreMesh(core_axis_name="c", subcore_axis_name="s")

    assert n_iter % 4 == 0 and n_iter >= 8

    @functools.partial(
        pl.kernel, mesh=mesh,
        out_type=jax.ShapeDtypeStruct((b, D_MODEL), jnp.float32),
        scratch_types=[
            pltpu.VMEM((2, 4 * _CHUNK), jnp.int32),
            pltpu.VMEM((2, 4 * _CHUNK), jnp.int32),
            pltpu.VMEM((2, _CHUNK, D_MODEL), jnp.float32),
            pltpu.VMEM((2, _CHUNK, D_MODEL), jnp.float32),
            pltpu.VMEM((2, _CHUNK, D_MODEL), jnp.float32),
            pltpu.VMEM_SHARED((1000, D_MODEL), jnp.float32),
            pltpu.SemaphoreType.DMA((2,)),
            pltpu.SemaphoreType.DMA((2,)),
            pltpu.SemaphoreType.DMA((2,)),
            pltpu.SemaphoreType.DMA((2,)),
        ],
    )
    def k(te_hbm, tp_hbm, eid_hbm, pid_hbm, out_hbm, eixb, pixb, ebufs,
          pbufs, obufs, tp_sh, isem, esem, psem, wsem):
        wid = lax.axis_index("s") * 2 + lax.axis_index("c")
        base = wid * per_w

        @pl.when(lax.axis_index("s") == 0)
        def _stage():
            pltpu.sync_copy(tp_hbm, tp_sh)

        plsc.subcore_barrier()

        pltpu.sync_copy(eid_hbm.at[pl.ds(base, 4 * _CHUNK)], eixb.at[0])
        pltpu.sync_copy(pid_hbm.at[pl.ds(base, 4 * _CHUNK)], pixb.at[0])
        for i in (0, 1):
            sl = pl.ds(i * _CHUNK, _CHUNK)
            pltpu.async_copy(te_hbm.at[eixb.at[0, sl]], ebufs.at[i],
                             esem.at[i])
            pltpu.async_copy(tp_sh.at[pixb.at[0, sl]], pbufs.at[i],
                             psem.at[i])
        pltpu.async_copy(eid_hbm.at[pl.ds(base + 4 * _CHUNK, 4 * _CHUNK)],
                         eixb.at[1], isem.at[1])
        pltpu.async_copy(pid_hbm.at[pl.ds(base + 4 * _CHUNK, 4 * _CHUNK)],
                         pixb.at[1], isem.at[1])

        def quad(fq, carry):
            cur = fq % 2
            nxt = 1 - cur
            for sub in range(4):
                i = fq * 4 + sub
                bb = sub % 2
                sq = pl.ds(sub * _CHUNK, _CHUNK)

                pltpu.make_async_copy(te_hbm.at[eixb.at[cur, sq]],
                                      ebufs.at[bb], esem.at[bb]).wait()
                pltpu.make_async_copy(tp_sh.at[pixb.at[cur, sq]],
                                      pbufs.at[bb], psem.at[bb]).wait()

                @pl.when(i >= 2)
                def _wdrain():
                    pltpu.make_async_copy(
                        obufs.at[bb], out_hbm.at[pl.ds(base, _CHUNK)],
                        wsem.at[bb]).wait()

                if sub == 3:
                    @pl.when(i + 5 < n_iter)
                    def _pfidx():
                        offb = base + (fq + 2) * 4 * _CHUNK
                        pltpu.async_copy(
                            eid_hbm.at[pl.ds(offb, 4 * _CHUNK)],
                            eixb.at[cur], isem.at[cur])
                        pltpu.async_copy(
                            pid_hbm.at[pl.ds(offb, 4 * _CHUNK)],
                            pixb.at[cur], isem.at[cur])

                def addrow(r4, c2):
                    for rr in range(4):
                        r2 = r4 * 4 + rr
                        for j in range(D_MODEL // 16):
                            sl = pl.ds(j * 16, 16)
                            obufs[bb, r2, sl] = (ebufs[bb, r2, sl]
                                                 + pbufs[bb, r2, sl])
                    return c2

                lax.fori_loop(0, _CHUNK // 4, addrow, 0)

                off = base + i * _CHUNK
                pltpu.async_copy(obufs.at[bb], out_hbm.at[pl.ds(off, _CHUNK)],
                                 wsem.at[bb])

                @pl.when(i + 2 < n_iter)
                def _gnext():
                    slot2 = nxt if sub >= 2 else cur
                    sq2 = pl.ds(((sub + 2) % 4) * _CHUNK, _CHUNK)
                    if sub == 2:
                        pltpu.make_async_copy(
                            eid_hbm.at[pl.ds(base, 4 * _CHUNK)],
                            eixb.at[slot2], isem.at[slot2]).wait()
                        pltpu.make_async_copy(
                            pid_hbm.at[pl.ds(base, 4 * _CHUNK)],
                            pixb.at[slot2], isem.at[slot2]).wait()
                    pltpu.async_copy(te_hbm.at[eixb.at[slot2, sq2]],
                                     ebufs.at[bb], esem.at[bb])
                    pltpu.async_copy(tp_sh.at[pixb.at[slot2, sq2]],
                                     pbufs.at[bb], psem.at[bb])
            return carry

        lax.fori_loop(0, n_iter // 4, quad, 0)

        for bb in (0, 1):
            pltpu.make_async_copy(obufs.at[bb],
                                  out_hbm.at[pl.ds(base, _CHUNK)],
                                  wsem.at[bb]).wait()

    return k(te, tp, eid, pid)


def kernel(element_ids, property_ids, element_table, property_table,
           fusion_w, fusion_b):
    bsz, n = element_ids.shape
    eid = element_ids.reshape(-1).astype(jnp.int32)
    pid = property_ids.reshape(-1).astype(jnp.int32)
    te, tp = _transform_tables(element_table, property_table, fusion_w,
                               fusion_b.reshape(1, -1))
    out = _sc_combine(te, tp, eid, pid)
    return out.reshape(bsz, n, D_MODEL)
